# baseline (speedup 1.0000x reference)
import functools

import jax
import jax.numpy as jnp
from jax import lax
from jax.experimental import pallas as pl
from jax.experimental.pallas import tpu as pltpu
from jax.experimental.pallas import tpu_sc as plsc

DT = 0.1

_NC = 2
_NS = 16
_NW = _NC * _NS

_HPAD = 256
_HCHUNK = 8192
_GCHUNK = 8192


def _f_sys(x, u):
    return jnp.stack([
        x[0] + u[0] * jnp.cos(x[2]) * DT,
        x[1] + u[0] * jnp.sin(x[2]) * DT,
        x[2] + u[1] * DT,
    ])


def _h_sys(x):
    return jnp.sqrt(x[0] ** 2 + x[1] ** 2)[None]


def _count_kernel(c_ref, rrow_ref, r_ref, k_ref, *, n_total):
    c = c_ref[...]
    rows = c.shape[0]
    t = c
    coli = lax.broadcasted_iota(jnp.int32, (rows, 128), 1)
    r2 = lax.broadcasted_iota(jnp.int32, (128, 128), 0)
    c2 = lax.broadcasted_iota(jnp.int32, (128, 128), 1)
    for b in range(7):
        half = 1 << b
        sel = (r2 == (((c2 >> b) | 1) << b)).astype(jnp.float32)
        perm = jnp.dot(c, sel, precision=lax.Precision.HIGHEST,
                       preferred_element_type=jnp.float32)
        maskb = (coli & half) == 0
        t = jnp.where(maskb, jnp.minimum(t, perm), t)
    c = jnp.minimum(t, rrow_ref[...])
    r = r_ref[0, 0]
    inv_n = jnp.float32(1.0 / n_total)
    x = (c - r) * jnp.float32(n_total)
    x = jnp.clip(x, jnp.float32(-2.0), jnp.float32(n_total + 8))
    est = jnp.floor(x).astype(jnp.int32) + 1
    base = jnp.clip(est - 4, 0, n_total)
    k = base
    for t in range(8):
        m = base + t
        u_m = r + m.astype(jnp.float32) * inv_n
        k = k + ((u_m <= c) & (m < n_total)).astype(jnp.int32)
    k_ref[...] = k


def _hist_body(n, counts_hbm, out_hbm, a_sh, idx_v, one_v, z_v):
    c = lax.axis_index("c")
    s = lax.axis_index("s")
    per_worker = n // _NW
    wch = n // _NS

    def _fill_ones(i, _):
        one_v[pl.ds(i * 16, 16)] = jnp.ones((16,), jnp.int32)
        return 0
    lax.fori_loop(0, _HCHUNK // 16, _fill_ones, 0)

    def _fill_zeros(i, _):
        z_v[pl.ds(i * 16, 16)] = jnp.zeros((16,), jnp.int32)
        return 0
    lax.fori_loop(0, _HCHUNK // 16, _fill_zeros, 0)

    for t in range(wch // _HCHUNK):
        pltpu.sync_copy(z_v, a_sh.at[pl.ds(s * wch + t * _HCHUNK, _HCHUNK)])

    @pl.when(s == 0)
    def _():
        pltpu.sync_copy(z_v.at[pl.ds(0, _HPAD)], a_sh.at[pl.ds(n, _HPAD)])

    plsc.subcore_barrier()

    base = (c * _NS + s) * per_worker
    for t in range(per_worker // _HCHUNK):
        pltpu.sync_copy(counts_hbm.at[pl.ds(base + t * _HCHUNK, _HCHUNK)],
                        idx_v)
        pltpu.sync_copy(one_v, a_sh.at[idx_v], add=True)
    plsc.subcore_barrier()

    pltpu.sync_copy(a_sh.at[pl.ds(s * wch, wch)],
                    out_hbm.at[c, pl.ds(s * wch, wch)])


def _scan_kernel(h0_ref, h1_ref, idx_ref, carry_ref, *, n_total):
    i = pl.program_id(0)
    a = (h0_ref[0] + h1_ref[0]).astype(jnp.float32)
    rows = lax.broadcasted_iota(jnp.int32, (1024, 1024), 0)
    cols = lax.broadcasted_iota(jnp.int32, (1024, 1024), 1)
    tri = (rows <= cols).astype(jnp.float32)
    crows = jnp.dot(a, tri, precision=lax.Precision.HIGHEST,
                    preferred_element_type=jnp.float32)
    s = crows[:, 1023:1024]
    r2 = lax.broadcasted_iota(jnp.int32, (128, 128), 0)
    c2 = lax.broadcasted_iota(jnp.int32, (128, 128), 1)
    strict = (c2 < r2).astype(jnp.float32)
    pref = jnp.dot(strict, s, precision=lax.Precision.HIGHEST,
                   preferred_element_type=jnp.float32)

    @pl.when(i == 0)
    def _():
        carry_ref[0] = 0.0

    carry = carry_ref[0]
    f = crows + pref + carry
    idx_ref[...] = jnp.minimum(f, n_total - 1).astype(jnp.int32)
    carry_ref[0] = carry + jnp.sum(s)


def _gather_body(n, flat_hbm, idx_hbm, out_hbm, idx_v, idx3_v, rows_v, sem):
    c = lax.axis_index("c")
    s = lax.axis_index("s")
    per_worker = n // _NW
    base = (c * _NS + s) * per_worker
    for t in range(per_worker // _GCHUNK):
        o = base + t * _GCHUNK
        pltpu.sync_copy(idx_hbm.at[pl.ds(o, _GCHUNK)], idx_v)

        def _expand(i, _):
            v3 = idx_v[pl.ds(i * 16, 16)] * 3
            idx3_v[pl.ds(i * 16, 16)] = v3
            idx3_v[pl.ds(_GCHUNK + i * 16, 16)] = v3 + 1
            idx3_v[pl.ds(2 * _GCHUNK + i * 16, 16)] = v3 + 2
            return 0
        lax.fori_loop(0, _GCHUNK // 16, _expand, 0)
        pltpu.async_copy(flat_hbm.at[idx3_v], rows_v, sem).wait()
        for k in range(3):
            pltpu.sync_copy(rows_v.at[pl.ds(k * _GCHUNK, _GCHUNK)],
                            out_hbm.at[pl.ds(k * n + o, _GCHUNK)])


def kernel(particles, u, z, M, Q_val, key):
    n = particles.shape[0]
    key_motion, key_resample = jax.random.split(key)
    u_noise = jax.random.multivariate_normal(
        key_motion, jnp.zeros_like(u), M, shape=(n,))
    u_particles = u + u_noise
    particles_bar = jax.vmap(_f_sys, in_axes=(0, 0))(particles, u_particles)
    z_expected = jax.vmap(_h_sys, in_axes=(0,))(particles_bar)
    innov = (z - z_expected).reshape(-1)
    weights = jnp.exp(-0.5 * innov ** 2 / Q_val)
    weights = weights + 1e-08
    weights = weights / jnp.sum(weights)
    r = jax.random.uniform(key_resample, minval=0.0, maxval=1.0 / n)
    c = jnp.cumsum(weights)

    rows = n // 128
    block_rows = rows // 16
    col0 = c.reshape(rows, 128)[:, 0]
    rrow = jnp.full((rows,), jnp.inf, jnp.float32)
    for kbit in range(13):
        blk = 1 << (kbit + 1)
        half = 1 << kbit
        mids = col0.reshape(rows // blk, blk)[:, half]
        rr = rrow.reshape(rows // blk, blk)
        rrow = jnp.concatenate(
            [jnp.minimum(rr[:, :half], mids[:, None]), rr[:, half:]],
            axis=1).reshape(-1)

    counts = pl.pallas_call(
        functools.partial(_count_kernel, n_total=n),
        grid=(16,),
        in_specs=[
            pl.BlockSpec((block_rows, 128), lambda i: (i, 0)),
            pl.BlockSpec((block_rows, 1), lambda i: (i, 0)),
            pl.BlockSpec((1, 1), lambda i: (0, 0)),
        ],
        out_specs=pl.BlockSpec((block_rows, 128), lambda i: (i, 0)),
        out_shape=jax.ShapeDtypeStruct((rows, 128), jnp.int32),
    )(c.reshape(rows, 128), rrow.reshape(rows, 1), r.reshape(1, 1))
    counts = counts.reshape(-1)

    mesh = plsc.VectorSubcoreMesh(core_axis_name="c", subcore_axis_name="s")
    hist2 = pl.kernel(
        functools.partial(_hist_body, n),
        out_type=jax.ShapeDtypeStruct((_NC, n), jnp.int32),
        mesh=mesh,
        scratch_types=[
            pltpu.VMEM_SHARED(((n + _HPAD),), jnp.int32),
            pltpu.VMEM((_HCHUNK,), jnp.int32),
            pltpu.VMEM((_HCHUNK,), jnp.int32),
            pltpu.VMEM((_HCHUNK,), jnp.int32),
        ],
    )(counts)

    side = 1024
    h3 = hist2.reshape(_NC, side, side)
    indices = pl.pallas_call(
        functools.partial(_scan_kernel, n_total=n),
        grid=(8,),
        in_specs=[
            pl.BlockSpec((1, 128, side), lambda i: (0, i, 0)),
            pl.BlockSpec((1, 128, side), lambda i: (1, i, 0)),
        ],
        out_specs=pl.BlockSpec((128, side), lambda i: (i, 0)),
        out_shape=jax.ShapeDtypeStruct((side, side), jnp.int32),
        scratch_shapes=[pltpu.SMEM((1,), jnp.float32)],
    )(h3, h3)
    indices = indices.reshape(-1)

    resampled_cols = pl.kernel(
        functools.partial(_gather_body, n),
        out_type=jax.ShapeDtypeStruct((3 * n,), jnp.float32),
        mesh=plsc.VectorSubcoreMesh(core_axis_name="c",
                                    subcore_axis_name="s"),
        scratch_types=[
            pltpu.VMEM((_GCHUNK,), jnp.int32),
            pltpu.VMEM((3 * _GCHUNK,), jnp.int32),
            pltpu.VMEM((3 * _GCHUNK,), jnp.float32),
            pltpu.SemaphoreType.DMA,
        ],
    )(particles_bar.reshape(-1), indices)
    particles_resampled = resampled_cols.reshape(3, n).T

    mean_pos = jnp.mean(particles_resampled[:, :2], axis=0)
    mean_theta = jnp.arctan2(
        jnp.mean(jnp.sin(particles_resampled[:, 2])),
        jnp.mean(jnp.cos(particles_resampled[:, 2])),
    )
    mu_now = jnp.array([mean_pos[0], mean_pos[1], mean_theta])
    return (mu_now, particles_resampled)


# speedup vs baseline: 2.2433x; 2.2433x over previous
import functools

import jax
import jax.numpy as jnp
from jax import lax
from jax.experimental import pallas as pl
from jax.experimental.pallas import tpu as pltpu
from jax.experimental.pallas import tpu_sc as plsc

DT = 0.1

_NC = 2
_NS = 16
_NW = _NC * _NS

_HPAD = 256
_HCHUNK = 8192
_GCHUNK = 8192


def _f_sys(x, u):
    return jnp.stack([
        x[0] + u[0] * jnp.cos(x[2]) * DT,
        x[1] + u[0] * jnp.sin(x[2]) * DT,
        x[2] + u[1] * DT,
    ])


def _h_sys(x):
    return jnp.sqrt(x[0] ** 2 + x[1] ** 2)[None]


def _count_kernel(c_ref, rrow_ref, r_ref, k_ref, *, n_total):
    c = c_ref[...]
    rows = c.shape[0]
    t = c
    coli = lax.broadcasted_iota(jnp.int32, (rows, 128), 1)
    r2 = lax.broadcasted_iota(jnp.int32, (128, 128), 0)
    c2 = lax.broadcasted_iota(jnp.int32, (128, 128), 1)
    for b in range(7):
        half = 1 << b
        sel = (r2 == (((c2 >> b) | 1) << b)).astype(jnp.float32)
        perm = jnp.dot(c, sel, precision=lax.Precision.HIGHEST,
                       preferred_element_type=jnp.float32)
        maskb = (coli & half) == 0
        t = jnp.where(maskb, jnp.minimum(t, perm), t)
    c = jnp.minimum(t, rrow_ref[...])
    r = r_ref[0, 0]
    inv_n = jnp.float32(1.0 / n_total)
    x = (c - r) * jnp.float32(n_total)
    x = jnp.clip(x, jnp.float32(-2.0), jnp.float32(n_total + 8))
    est = jnp.floor(x).astype(jnp.int32) + 1
    base = jnp.clip(est - 4, 0, n_total)
    k = base
    for t in range(8):
        m = base + t
        u_m = r + m.astype(jnp.float32) * inv_n
        k = k + ((u_m <= c) & (m < n_total)).astype(jnp.int32)
    k_ref[...] = k


def _hist_body(n, counts_hbm, out_hbm, a_sh, idx_v, one_v, z_v):
    c = lax.axis_index("c")
    s = lax.axis_index("s")
    per_worker = n // _NW
    wch = n // _NS

    def _fill_ones(i, _):
        one_v[pl.ds(i * 16, 16)] = jnp.ones((16,), jnp.int32)
        return 0
    lax.fori_loop(0, _HCHUNK // 16, _fill_ones, 0)

    def _fill_zeros(i, _):
        z_v[pl.ds(i * 16, 16)] = jnp.zeros((16,), jnp.int32)
        return 0
    lax.fori_loop(0, _HCHUNK // 16, _fill_zeros, 0)

    for t in range(wch // _HCHUNK):
        pltpu.sync_copy(z_v, a_sh.at[pl.ds(s * wch + t * _HCHUNK, _HCHUNK)])

    @pl.when(s == 0)
    def _():
        pltpu.sync_copy(z_v.at[pl.ds(0, _HPAD)], a_sh.at[pl.ds(n, _HPAD)])

    plsc.subcore_barrier()

    base = (c * _NS + s) * per_worker
    for t in range(per_worker // _HCHUNK):
        pltpu.sync_copy(counts_hbm.at[pl.ds(base + t * _HCHUNK, _HCHUNK)],
                        idx_v)
        pltpu.sync_copy(one_v, a_sh.at[idx_v], add=True)
    plsc.subcore_barrier()

    pltpu.sync_copy(a_sh.at[pl.ds(s * wch, wch)],
                    out_hbm.at[c, pl.ds(s * wch, wch)])


def _scan_kernel(h0_ref, h1_ref, idx_ref, carry_ref, *, n_total):
    i = pl.program_id(0)
    a = (h0_ref[0] + h1_ref[0]).astype(jnp.float32)
    rows = lax.broadcasted_iota(jnp.int32, (1024, 1024), 0)
    cols = lax.broadcasted_iota(jnp.int32, (1024, 1024), 1)
    tri = (rows <= cols).astype(jnp.float32)
    crows = jnp.dot(a, tri, precision=lax.Precision.HIGHEST,
                    preferred_element_type=jnp.float32)
    s = crows[:, 1023:1024]
    r2 = lax.broadcasted_iota(jnp.int32, (128, 128), 0)
    c2 = lax.broadcasted_iota(jnp.int32, (128, 128), 1)
    strict = (c2 < r2).astype(jnp.float32)
    pref = jnp.dot(strict, s, precision=lax.Precision.HIGHEST,
                   preferred_element_type=jnp.float32)

    @pl.when(i == 0)
    def _():
        carry_ref[0] = 0.0

    carry = carry_ref[0]
    f = crows + pref + carry
    idx_ref[...] = jnp.minimum(f, n_total - 1).astype(jnp.int32)
    carry_ref[0] = carry + jnp.sum(s)


def _gather_body(n, c0_hbm, c1_hbm, c2_hbm, idx_hbm, out_hbm, idx_v, rows_v,
                 sem):
    c = lax.axis_index("c")
    s = lax.axis_index("s")
    per_worker = n // _NW
    base = (c * _NS + s) * per_worker
    for t in range(per_worker // _GCHUNK):
        o = base + t * _GCHUNK
        pltpu.sync_copy(idx_hbm.at[pl.ds(o, _GCHUNK)], idx_v)
        copies = [
            pltpu.async_copy(col.at[idx_v],
                             rows_v.at[pl.ds(k * _GCHUNK, _GCHUNK)], sem)
            for k, col in enumerate((c0_hbm, c1_hbm, c2_hbm))
        ]
        for cp in copies:
            cp.wait()
        for k in range(3):
            pltpu.sync_copy(rows_v.at[pl.ds(k * _GCHUNK, _GCHUNK)],
                            out_hbm.at[pl.ds(k * n + o, _GCHUNK)])


def kernel(particles, u, z, M, Q_val, key):
    n = particles.shape[0]
    key_motion, key_resample = jax.random.split(key)
    u_noise = jax.random.multivariate_normal(
        key_motion, jnp.zeros_like(u), M, shape=(n,))
    u_particles = u + u_noise
    particles_bar = jax.vmap(_f_sys, in_axes=(0, 0))(particles, u_particles)
    z_expected = jax.vmap(_h_sys, in_axes=(0,))(particles_bar)
    innov = (z - z_expected).reshape(-1)
    weights = jnp.exp(-0.5 * innov ** 2 / Q_val)
    weights = weights + 1e-08
    weights = weights / jnp.sum(weights)
    r = jax.random.uniform(key_resample, minval=0.0, maxval=1.0 / n)
    c = jnp.cumsum(weights)

    rows = n // 128
    block_rows = rows // 16
    col0 = c.reshape(rows, 128)[:, 0]
    rrow = jnp.full((rows,), jnp.inf, jnp.float32)
    for kbit in range(13):
        blk = 1 << (kbit + 1)
        half = 1 << kbit
        mids = col0.reshape(rows // blk, blk)[:, half]
        rr = rrow.reshape(rows // blk, blk)
        rrow = jnp.concatenate(
            [jnp.minimum(rr[:, :half], mids[:, None]), rr[:, half:]],
            axis=1).reshape(-1)

    counts = pl.pallas_call(
        functools.partial(_count_kernel, n_total=n),
        grid=(16,),
        in_specs=[
            pl.BlockSpec((block_rows, 128), lambda i: (i, 0)),
            pl.BlockSpec((block_rows, 1), lambda i: (i, 0)),
            pl.BlockSpec((1, 1), lambda i: (0, 0)),
        ],
        out_specs=pl.BlockSpec((block_rows, 128), lambda i: (i, 0)),
        out_shape=jax.ShapeDtypeStruct((rows, 128), jnp.int32),
    )(c.reshape(rows, 128), rrow.reshape(rows, 1), r.reshape(1, 1))
    counts = counts.reshape(-1)

    mesh = plsc.VectorSubcoreMesh(core_axis_name="c", subcore_axis_name="s")
    hist2 = pl.kernel(
        functools.partial(_hist_body, n),
        out_type=jax.ShapeDtypeStruct((_NC, n), jnp.int32),
        mesh=mesh,
        scratch_types=[
            pltpu.VMEM_SHARED(((n + _HPAD),), jnp.int32),
            pltpu.VMEM((_HCHUNK,), jnp.int32),
            pltpu.VMEM((_HCHUNK,), jnp.int32),
            pltpu.VMEM((_HCHUNK,), jnp.int32),
        ],
    )(counts)

    side = 1024
    h3 = hist2.reshape(_NC, side, side)
    indices = pl.pallas_call(
        functools.partial(_scan_kernel, n_total=n),
        grid=(8,),
        in_specs=[
            pl.BlockSpec((1, 128, side), lambda i: (0, i, 0)),
            pl.BlockSpec((1, 128, side), lambda i: (1, i, 0)),
        ],
        out_specs=pl.BlockSpec((128, side), lambda i: (i, 0)),
        out_shape=jax.ShapeDtypeStruct((side, side), jnp.int32),
        scratch_shapes=[pltpu.SMEM((1,), jnp.float32)],
    )(h3, h3)
    indices = indices.reshape(-1)

    resampled_cols = pl.kernel(
        functools.partial(_gather_body, n),
        out_type=jax.ShapeDtypeStruct((3 * n,), jnp.float32),
        mesh=plsc.VectorSubcoreMesh(core_axis_name="c",
                                    subcore_axis_name="s"),
        scratch_types=[
            pltpu.VMEM((_GCHUNK,), jnp.int32),
            pltpu.VMEM((3 * _GCHUNK,), jnp.float32),
            pltpu.SemaphoreType.DMA,
        ],
    )(particles_bar[:, 0], particles_bar[:, 1], particles_bar[:, 2],
      indices)
    particles_resampled = resampled_cols.reshape(3, n).T

    mean_pos = jnp.mean(particles_resampled[:, :2], axis=0)
    mean_theta = jnp.arctan2(
        jnp.mean(jnp.sin(particles_resampled[:, 2])),
        jnp.mean(jnp.cos(particles_resampled[:, 2])),
    )
    mu_now = jnp.array([mean_pos[0], mean_pos[1], mean_theta])
    return (mu_now, particles_resampled)


# speedup vs baseline: 2.3975x; 1.0687x over previous
import functools

import jax
import jax.numpy as jnp
from jax import lax
from jax.experimental import pallas as pl
from jax.experimental.pallas import tpu as pltpu
from jax.experimental.pallas import tpu_sc as plsc

DT = 0.1

_NC = 2
_NS = 16
_NW = _NC * _NS

_HPAD = 256
_HCHUNK = 8192
_GCHUNK = 8192


def _f_sys(x, u):
    return jnp.stack([
        x[0] + u[0] * jnp.cos(x[2]) * DT,
        x[1] + u[0] * jnp.sin(x[2]) * DT,
        x[2] + u[1] * DT,
    ])


def _h_sys(x):
    return jnp.sqrt(x[0] ** 2 + x[1] ** 2)[None]


def _count_kernel(c_ref, rrow_ref, r_ref, k_ref, *, n_total):
    c = c_ref[...]
    rows = c.shape[0]
    t = c
    coli = lax.broadcasted_iota(jnp.int32, (rows, 128), 1)
    r2 = lax.broadcasted_iota(jnp.int32, (128, 128), 0)
    c2 = lax.broadcasted_iota(jnp.int32, (128, 128), 1)
    for b in range(7):
        half = 1 << b
        sel = (r2 == (((c2 >> b) | 1) << b)).astype(jnp.float32)
        perm = jnp.dot(c, sel, precision=lax.Precision.HIGHEST,
                       preferred_element_type=jnp.float32)
        maskb = (coli & half) == 0
        t = jnp.where(maskb, jnp.minimum(t, perm), t)
    c = jnp.minimum(t, rrow_ref[...])
    r = r_ref[0, 0]
    inv_n = jnp.float32(1.0 / n_total)
    x = (c - r) * jnp.float32(n_total)
    x = jnp.clip(x, jnp.float32(-2.0), jnp.float32(n_total + 8))
    est = jnp.floor(x).astype(jnp.int32) + 1
    base = jnp.clip(est - 4, 0, n_total)
    k = base
    for t in range(8):
        m = base + t
        u_m = r + m.astype(jnp.float32) * inv_n
        k = k + ((u_m <= c) & (m < n_total)).astype(jnp.int32)
    k_ref[...] = k


def _hist_body(n, counts_hbm, out_hbm, a_sh, idx_v, one_v, z_v):
    c = lax.axis_index("c")
    s = lax.axis_index("s")
    per_worker = n // _NW
    wch = n // _NS

    def _fill_ones(i, _):
        one_v[pl.ds(i * 16, 16)] = jnp.ones((16,), jnp.int32)
        return 0
    lax.fori_loop(0, _HCHUNK // 16, _fill_ones, 0)

    def _fill_zeros(i, _):
        z_v[pl.ds(i * 16, 16)] = jnp.zeros((16,), jnp.int32)
        return 0
    lax.fori_loop(0, _HCHUNK // 16, _fill_zeros, 0)

    for t in range(wch // _HCHUNK):
        pltpu.sync_copy(z_v, a_sh.at[pl.ds(s * wch + t * _HCHUNK, _HCHUNK)])

    @pl.when(s == 0)
    def _():
        pltpu.sync_copy(z_v.at[pl.ds(0, _HPAD)], a_sh.at[pl.ds(n, _HPAD)])

    plsc.subcore_barrier()

    base = (c * _NS + s) * per_worker
    for t in range(per_worker // _HCHUNK):
        pltpu.sync_copy(counts_hbm.at[pl.ds(base + t * _HCHUNK, _HCHUNK)],
                        idx_v)
        pltpu.sync_copy(one_v, a_sh.at[idx_v], add=True)
    plsc.subcore_barrier()

    pltpu.sync_copy(a_sh.at[pl.ds(s * wch, wch)],
                    out_hbm.at[c, pl.ds(s * wch, wch)])


def _scan_kernel(h0_ref, h1_ref, idx_ref, carry_ref, *, n_total):
    i = pl.program_id(0)
    a = (h0_ref[0] + h1_ref[0]).astype(jnp.float32)
    rows = lax.broadcasted_iota(jnp.int32, (1024, 1024), 0)
    cols = lax.broadcasted_iota(jnp.int32, (1024, 1024), 1)
    tri = (rows <= cols).astype(jnp.float32)
    crows = jnp.dot(a, tri, precision=lax.Precision.HIGHEST,
                    preferred_element_type=jnp.float32)
    s = crows[:, 1023:1024]
    r2 = lax.broadcasted_iota(jnp.int32, (128, 128), 0)
    c2 = lax.broadcasted_iota(jnp.int32, (128, 128), 1)
    strict = (c2 < r2).astype(jnp.float32)
    pref = jnp.dot(strict, s, precision=lax.Precision.HIGHEST,
                   preferred_element_type=jnp.float32)

    @pl.when(i == 0)
    def _():
        carry_ref[0] = 0.0

    carry = carry_ref[0]
    f = crows + pref + carry
    idx_ref[...] = jnp.minimum(f, n_total - 1).astype(jnp.int32)
    carry_ref[0] = carry + jnp.sum(s)


def _mu_kernel(x_ref, out_ref):
    x = x_ref[...]
    s_plain = jnp.sum(x)
    s_sin = jnp.sum(jnp.sin(x))
    s_cos = jnp.sum(jnp.cos(x))
    lanei = lax.broadcasted_iota(jnp.int32, (1, 128), 1)
    v = jnp.where(lanei == 0, s_plain,
                  jnp.where(lanei == 1, s_sin,
                            jnp.where(lanei == 2, s_cos,
                                      jnp.float32(0.0))))
    out_ref[0] = v


def _gather_body(n, c0_hbm, c1_hbm, c2_hbm, idx_hbm, out_hbm, idx_v, rows_v,
                 sem):
    c = lax.axis_index("c")
    s = lax.axis_index("s")
    per_worker = n // _NW
    base = (c * _NS + s) * per_worker
    for t in range(per_worker // _GCHUNK):
        o = base + t * _GCHUNK
        pltpu.sync_copy(idx_hbm.at[pl.ds(o, _GCHUNK)], idx_v)
        copies = [
            pltpu.async_copy(col.at[idx_v],
                             rows_v.at[pl.ds(k * _GCHUNK, _GCHUNK)], sem)
            for k, col in enumerate((c0_hbm, c1_hbm, c2_hbm))
        ]
        for cp in copies:
            cp.wait()
        for k in range(3):
            pltpu.sync_copy(rows_v.at[pl.ds(k * _GCHUNK, _GCHUNK)],
                            out_hbm.at[pl.ds(k * n + o, _GCHUNK)])


def kernel(particles, u, z, M, Q_val, key):
    n = particles.shape[0]
    key_motion, key_resample = jax.random.split(key)
    u_noise = jax.random.multivariate_normal(
        key_motion, jnp.zeros_like(u), M, shape=(n,))
    u_particles = u + u_noise
    particles_bar = jax.vmap(_f_sys, in_axes=(0, 0))(particles, u_particles)
    z_expected = jax.vmap(_h_sys, in_axes=(0,))(particles_bar)
    innov = (z - z_expected).reshape(-1)
    weights = jnp.exp(-0.5 * innov ** 2 / Q_val)
    weights = weights + 1e-08
    weights = weights / jnp.sum(weights)
    r = jax.random.uniform(key_resample, minval=0.0, maxval=1.0 / n)
    c = jnp.cumsum(weights)

    rows = n // 128
    block_rows = rows // 16
    col0 = c.reshape(rows, 128)[:, 0]
    rrow = jnp.full((rows,), jnp.inf, jnp.float32)
    for kbit in range(13):
        blk = 1 << (kbit + 1)
        half = 1 << kbit
        mids = col0.reshape(rows // blk, blk)[:, half]
        rr = rrow.reshape(rows // blk, blk)
        rrow = jnp.concatenate(
            [jnp.minimum(rr[:, :half], mids[:, None]), rr[:, half:]],
            axis=1).reshape(-1)

    counts = pl.pallas_call(
        functools.partial(_count_kernel, n_total=n),
        grid=(16,),
        in_specs=[
            pl.BlockSpec((block_rows, 128), lambda i: (i, 0)),
            pl.BlockSpec((block_rows, 1), lambda i: (i, 0)),
            pl.BlockSpec((1, 1), lambda i: (0, 0)),
        ],
        out_specs=pl.BlockSpec((block_rows, 128), lambda i: (i, 0)),
        out_shape=jax.ShapeDtypeStruct((rows, 128), jnp.int32),
    )(c.reshape(rows, 128), rrow.reshape(rows, 1), r.reshape(1, 1))
    counts = counts.reshape(-1)

    mesh = plsc.VectorSubcoreMesh(core_axis_name="c", subcore_axis_name="s")
    hist2 = pl.kernel(
        functools.partial(_hist_body, n),
        out_type=jax.ShapeDtypeStruct((_NC, n), jnp.int32),
        mesh=mesh,
        scratch_types=[
            pltpu.VMEM_SHARED(((n + _HPAD),), jnp.int32),
            pltpu.VMEM((_HCHUNK,), jnp.int32),
            pltpu.VMEM((_HCHUNK,), jnp.int32),
            pltpu.VMEM((_HCHUNK,), jnp.int32),
        ],
    )(counts)

    side = 1024
    h3 = hist2.reshape(_NC, side, side)
    indices = pl.pallas_call(
        functools.partial(_scan_kernel, n_total=n),
        grid=(8,),
        in_specs=[
            pl.BlockSpec((1, 128, side), lambda i: (0, i, 0)),
            pl.BlockSpec((1, 128, side), lambda i: (1, i, 0)),
        ],
        out_specs=pl.BlockSpec((128, side), lambda i: (i, 0)),
        out_shape=jax.ShapeDtypeStruct((side, side), jnp.int32),
        scratch_shapes=[pltpu.SMEM((1,), jnp.float32)],
    )(h3, h3)
    indices = indices.reshape(-1)

    resampled_cols = pl.kernel(
        functools.partial(_gather_body, n),
        out_type=jax.ShapeDtypeStruct((3 * n,), jnp.float32),
        mesh=plsc.VectorSubcoreMesh(core_axis_name="c",
                                    subcore_axis_name="s"),
        scratch_types=[
            pltpu.VMEM((_GCHUNK,), jnp.int32),
            pltpu.VMEM((3 * _GCHUNK,), jnp.float32),
            pltpu.SemaphoreType.DMA,
        ],
    )(particles_bar[:, 0], particles_bar[:, 1], particles_bar[:, 2],
      indices)
    particles_resampled = resampled_cols.reshape(3, n).T

    mrows = (3 * n) // 128
    parts = pl.pallas_call(
        _mu_kernel,
        grid=(24,),
        in_specs=[pl.BlockSpec((1024, 128), lambda i: (i, 0))],
        out_specs=pl.BlockSpec((1, 1, 128), lambda i: (i, 0, 0)),
        out_shape=jax.ShapeDtypeStruct((24, 1, 128), jnp.float32),
    )(resampled_cols.reshape(mrows, 128))
    npf = jnp.float32(n)
    sum_px = jnp.sum(parts[0:8, 0, 0])
    sum_py = jnp.sum(parts[8:16, 0, 0])
    ssin = jnp.sum(parts[16:24, 0, 1])
    scos = jnp.sum(parts[16:24, 0, 2])
    mean_theta = jnp.arctan2(ssin / npf, scos / npf)
    mu_now = jnp.array([sum_px / npf, sum_py / npf, mean_theta])
    return (mu_now, particles_resampled)
